# baseline (speedup 1.0000x reference)
import jax
from jax.experimental import pallas as pl
from jax.experimental.pallas import tpu as pltpu


def _copy_body(pe_ref, out_ref, sem):
    seq_len = out_ref.shape[1]
    copy = pltpu.make_async_copy(
        pe_ref.at[:, pl.ds(0, seq_len), :], out_ref, sem
    )
    copy.start()
    copy.wait()


def kernel(x, pe):
    seq_len = x.shape[1]
    out_shape = jax.ShapeDtypeStruct((1, seq_len, pe.shape[2]), pe.dtype)
    return pl.pallas_call(
        _copy_body,
        out_shape=out_shape,
        in_specs=[pl.BlockSpec(memory_space=pl.ANY)],
        out_specs=pl.BlockSpec(memory_space=pl.ANY),
        scratch_shapes=[pltpu.SemaphoreType.DMA],
    )(pe)


# speedup vs baseline: 1.0001x; 1.0001x over previous
import jax
from jax.experimental import pallas as pl
from jax.experimental.pallas import tpu as pltpu


_N_CHUNKS = 8


def _copy_body(pe_ref, out_ref, sems):
    seq_len = out_ref.shape[1]
    chunk = seq_len // _N_CHUNKS
    copies = []
    for i in range(_N_CHUNKS):
        lo = i * chunk
        copies.append(pltpu.make_async_copy(
            pe_ref.at[:, pl.ds(lo, chunk), :],
            out_ref.at[:, pl.ds(lo, chunk), :],
            sems.at[i],
        ))
    for c in copies:
        c.start()
    for c in copies:
        c.wait()


def kernel(x, pe):
    seq_len = x.shape[1]
    out_shape = jax.ShapeDtypeStruct((1, seq_len, pe.shape[2]), pe.dtype)
    return pl.pallas_call(
        _copy_body,
        out_shape=out_shape,
        in_specs=[pl.BlockSpec(memory_space=pl.ANY)],
        out_specs=pl.BlockSpec(memory_space=pl.ANY),
        scratch_shapes=[pltpu.SemaphoreType.DMA((_N_CHUNKS,))],
    )(pe)


# speedup vs baseline: 33.5231x; 33.5190x over previous
import jax
from jax.experimental import pallas as pl
from jax.experimental.pallas import tpu as pltpu

_BLOCK_ROWS = 512


def _copy_body(pe_ref, out_ref):
    out_ref[...] = pe_ref[...]


def kernel(x, pe):
    seq_len = x.shape[1]
    d_model = pe.shape[2]
    grid = (seq_len // _BLOCK_ROWS,)
    out_shape = jax.ShapeDtypeStruct((1, seq_len, d_model), pe.dtype)
    return pl.pallas_call(
        _copy_body,
        grid=grid,
        in_specs=[pl.BlockSpec((1, _BLOCK_ROWS, d_model), lambda i: (0, i, 0))],
        out_specs=pl.BlockSpec((1, _BLOCK_ROWS, d_model), lambda i: (0, i, 0)),
        out_shape=out_shape,
    )(pe)
